# Initial kernel scaffold; baseline (speedup 1.0000x reference)
import jax
import jax.numpy as jnp
from jax.experimental import pallas as pl


def kernel(x):
    raise NotImplementedError("write your pallas kernel here")


# speedup vs baseline: 33.5967x; 33.5967x over previous
import functools
import math

import jax
import jax.numpy as jnp
from jax.experimental import pallas as pl
from jax.experimental.pallas import tpu as pltpu

_LENGTH = 5
_SIGMA = 0.5
_VMEM_LIMIT = 64 * 1024 * 1024


def _gaussian_weights_1d(L, sigma):
    lim = (L - 1.0) / 2.0
    g = [float(i) - lim for i in range(L)]
    w = [math.exp(-(v * v) / (2.0 * sigma * sigma)) for v in g]
    tot = sum(w)
    return tuple(v / tot for v in w)


def _fused_kernel(x_ref, o_ref, *, w1d, H, W, pad):
    x = x_ref[0]
    s = jnp.sum(x, axis=0)

    Wp = W + 2 * pad
    zw = jnp.zeros((H, pad), jnp.float32)
    zh = jnp.zeros((pad, Wp), jnp.float32)
    sp = jnp.concatenate([zh, jnp.concatenate([zw, s, zw], axis=1), zh], axis=0)

    L = len(w1d)
    h = None
    for dw in range(L):
        term = w1d[dw] * sp[:, dw:dw + W]
        h = term if h is None else h + term
    g = None
    for dh in range(L):
        term = w1d[dh] * h[dh:dh + H, :]
        g = term if g is None else g + term

    o_ref[...] = jnp.broadcast_to(g[None, None], o_ref.shape).astype(o_ref.dtype)


def kernel(x):
    N, C, H, W = x.shape
    L = _LENGTH
    pad = L // 2
    w1d = _gaussian_weights_1d(L, _SIGMA)

    return pl.pallas_call(
        functools.partial(_fused_kernel, w1d=w1d, H=H, W=W, pad=pad),
        out_shape=jax.ShapeDtypeStruct((N, C, H, W), x.dtype),
        grid=(N,),
        in_specs=[pl.BlockSpec((1, C, H, W), lambda n: (n, 0, 0, 0))],
        out_specs=pl.BlockSpec((1, C, H, W), lambda n: (n, 0, 0, 0)),
        compiler_params=pltpu.CompilerParams(
            dimension_semantics=("parallel",),
            vmem_limit_bytes=_VMEM_LIMIT,
        ),
    )(x)


# speedup vs baseline: 39.2950x; 1.1696x over previous
import functools
import math

import jax
import jax.numpy as jnp
from jax.experimental import pallas as pl
from jax.experimental.pallas import tpu as pltpu

_LENGTH = 5
_SIGMA = 0.5
_VMEM_LIMIT = 64 * 1024 * 1024


def _gaussian_weights_1d(L, sigma):
    lim = (L - 1.0) / 2.0
    g = [float(i) - lim for i in range(L)]
    w = [math.exp(-(v * v) / (2.0 * sigma * sigma)) for v in g]
    tot = sum(w)
    return tuple(v / tot for v in w)


def _fused_kernel(x_ref, o_ref, *, w1d, H, W, pad, B):
    Wp = W + 2 * pad
    L = len(w1d)
    for b in range(B):
        s = jnp.sum(x_ref[b], axis=0)

        zw = jnp.zeros((H, pad), jnp.float32)
        zh = jnp.zeros((pad, Wp), jnp.float32)
        sp = jnp.concatenate(
            [zh, jnp.concatenate([zw, s, zw], axis=1), zh], axis=0)

        h = None
        for dw in range(L):
            term = w1d[dw] * sp[:, dw:dw + W]
            h = term if h is None else h + term
        g = None
        for dh in range(L):
            term = w1d[dh] * h[dh:dh + H, :]
            g = term if g is None else g + term

        o_ref[b] = jnp.broadcast_to(g[None], o_ref.shape[1:]).astype(o_ref.dtype)


def kernel(x):
    N, C, H, W = x.shape
    L = _LENGTH
    pad = L // 2
    w1d = _gaussian_weights_1d(L, _SIGMA)

    B = 2
    return pl.pallas_call(
        functools.partial(_fused_kernel, w1d=w1d, H=H, W=W, pad=pad, B=B),
        out_shape=jax.ShapeDtypeStruct((N, C, H, W), x.dtype),
        grid=(N // B,),
        in_specs=[pl.BlockSpec((B, C, H, W), lambda n: (n, 0, 0, 0))],
        out_specs=pl.BlockSpec((B, C, H, W), lambda n: (n, 0, 0, 0)),
        compiler_params=pltpu.CompilerParams(
            dimension_semantics=("parallel",),
            vmem_limit_bytes=_VMEM_LIMIT,
        ),
    )(x)


# speedup vs baseline: 40.5221x; 1.0312x over previous
import functools
import math

import jax
import jax.numpy as jnp
from jax.experimental import pallas as pl
from jax.experimental.pallas import tpu as pltpu

_LENGTH = 5
_SIGMA = 0.5
_VMEM_LIMIT = 64 * 1024 * 1024


def _gaussian_weights_1d(L, sigma):
    lim = (L - 1.0) / 2.0
    g = [float(i) - lim for i in range(L)]
    w = [math.exp(-(v * v) / (2.0 * sigma * sigma)) for v in g]
    tot = sum(w)
    return tuple(v / tot for v in w)


def _fused_kernel(x_ref, o_ref, *, w1d, H, W, pad, B):
    Wp = W + 2 * pad
    L = len(w1d)
    for b in range(B):
        s = jnp.sum(x_ref[b], axis=0)

        zw = jnp.zeros((H, pad), jnp.float32)
        zh = jnp.zeros((pad, Wp), jnp.float32)
        sp = jnp.concatenate(
            [zh, jnp.concatenate([zw, s, zw], axis=1), zh], axis=0)

        h = None
        for dw in range(L):
            term = w1d[dw] * sp[:, dw:dw + W]
            h = term if h is None else h + term
        g = None
        for dh in range(L):
            term = w1d[dh] * h[dh:dh + H, :]
            g = term if g is None else g + term

        o_ref[b] = jnp.broadcast_to(g[None], o_ref.shape[1:]).astype(o_ref.dtype)


def kernel(x):
    N, C, H, W = x.shape
    L = _LENGTH
    pad = L // 2
    w1d = _gaussian_weights_1d(L, _SIGMA)

    B = 4
    return pl.pallas_call(
        functools.partial(_fused_kernel, w1d=w1d, H=H, W=W, pad=pad, B=B),
        out_shape=jax.ShapeDtypeStruct((N, C, H, W), x.dtype),
        grid=(N // B,),
        in_specs=[pl.BlockSpec((B, C, H, W), lambda n: (n, 0, 0, 0))],
        out_specs=pl.BlockSpec((B, C, H, W), lambda n: (n, 0, 0, 0)),
        compiler_params=pltpu.CompilerParams(
            dimension_semantics=("parallel",),
            vmem_limit_bytes=_VMEM_LIMIT,
        ),
    )(x)
